# baseline (speedup 1.0000x reference)
import functools

import jax
import jax.numpy as jnp
import numpy as np
from jax import lax
from jax.experimental import pallas as pl
from jax.experimental.pallas import tpu as pltpu
from jax.experimental.pallas import tpu_sc as plsc

IPC = 200
NUM_CLASSES = 100
CHANNEL, H, W = 3, 32, 32
N_PER_C = 10
DIM = CHANNEL * H * W
B = NUM_CLASSES * N_PER_C
ROWS = IPC * NUM_CLASSES

NC, NS = 2, 16
NW = NC * NS
CHUNK = 8


def _static_indices() -> np.ndarray:
    key = jax.random.key(42)
    parts = []
    for i in range(NUM_CLASSES):
        perm = jax.random.permutation(jax.random.fold_in(key, i), IPC)[:N_PER_C]
        parts.append(np.sort(np.asarray(perm)) + IPC * i)
    return np.concatenate(parts).astype(np.int32)


_INDICES = _static_indices()

_mesh = plsc.VectorSubcoreMesh(core_axis_name="c", subcore_axis_name="s")


def _make_gather(nvalid: int, npad: int):
    bpw = npad // NW
    nchunk = bpw // CHUNK

    @functools.partial(
        pl.kernel,
        mesh=_mesh,
        out_type=(
            jax.ShapeDtypeStruct((nvalid, DIM), jnp.float32),
            jax.ShapeDtypeStruct((nvalid,), jnp.int32),
        ),
        scratch_types=[
            pltpu.VMEM((bpw,), jnp.int32),
            pltpu.VMEM((bpw, DIM), jnp.float32),
            pltpu.VMEM((bpw,), jnp.int32),
            pltpu.SemaphoreType.DMA,
            pltpu.SemaphoreType.DMA,
        ],
    )
    def _gather_sc(table, idx, labs, out, labs_out, idx_v, rows_v, labs_v,
                   gsem, lsem):
        wid = lax.axis_index("s") * NC + lax.axis_index("c")
        base = wid * bpw
        pltpu.sync_copy(idx.at[pl.ds(base, bpw)], idx_v)
        rows_cp = pltpu.async_copy(table.at[idx_v], rows_v, gsem)
        labs_cp = pltpu.async_copy(labs.at[idx_v], labs_v, lsem)
        rows_cp.wait()
        labs_cp.wait()
        for k in range(nchunk):
            off = base + k * CHUNK

            @pl.when(off < nvalid)
            def _(k=k, off=off):
                pltpu.sync_copy(rows_v.at[pl.ds(k * CHUNK, CHUNK)],
                                out.at[pl.ds(off, CHUNK)])
                pltpu.sync_copy(labs_v.at[pl.ds(k * CHUNK, CHUNK)],
                                labs_out.at[pl.ds(off, CHUNK)])

    return _gather_sc


_HALF = 512
_gather_a = _make_gather(_HALF, _HALF)
_gather_b = _make_gather(B - _HALF, _HALF)

_IDX_A = _INDICES[:_HALF]
_IDX_B = np.concatenate([_INDICES[_HALF:],
                         np.zeros(_HALF - (B - _HALF), np.int32)])


def kernel(placeholder, table, labs):
    out_a, labs_a = _gather_a(table, jnp.asarray(_IDX_A), labs)
    out_b, labs_b = _gather_b(table, jnp.asarray(_IDX_B), labs)
    imgs = jnp.concatenate([out_a, out_b]).reshape(B, CHANNEL, H, W)
    labs_out = jnp.concatenate([labs_a, labs_b])
    indices = jnp.asarray(_INDICES)
    return (imgs, labs_out, indices)


# speedup vs baseline: 1.3525x; 1.3525x over previous
import functools

import jax
import jax.numpy as jnp
import numpy as np
from jax import lax
from jax.experimental import pallas as pl
from jax.experimental.pallas import tpu as pltpu
from jax.experimental.pallas import tpu_sc as plsc

IPC = 200
NUM_CLASSES = 100
CHANNEL, H, W = 3, 32, 32
N_PER_C = 10
DIM = CHANNEL * H * W
B = NUM_CLASSES * N_PER_C
ROWS = IPC * NUM_CLASSES

NC, NS = 2, 16
NW = NC * NS
B_PAD = 1024
BPW = B_PAD // NW
HALF = BPW // 2
CHUNK = 8


def _static_indices() -> np.ndarray:
    key = jax.random.key(42)
    parts = []
    for i in range(NUM_CLASSES):
        perm = jax.random.permutation(jax.random.fold_in(key, i), IPC)[:N_PER_C]
        parts.append(np.sort(np.asarray(perm)) + IPC * i)
    return np.concatenate(parts).astype(np.int32)


_INDICES = _static_indices()
_IDX_PAD = np.concatenate([_INDICES, np.zeros(B_PAD - B, np.int32)])

_mesh = plsc.VectorSubcoreMesh(core_axis_name="c", subcore_axis_name="s")


@functools.partial(
    pl.kernel,
    mesh=_mesh,
    out_type=(
        jax.ShapeDtypeStruct((B, DIM), jnp.float32),
        jax.ShapeDtypeStruct((B,), jnp.int32),
    ),
    scratch_types=[
        pltpu.VMEM((BPW,), jnp.int32),
        pltpu.VMEM((BPW, DIM), jnp.float32),
        pltpu.VMEM((BPW,), jnp.int32),
        pltpu.SemaphoreType.DMA,
        pltpu.SemaphoreType.DMA,
        pltpu.SemaphoreType.DMA,
        pltpu.SemaphoreType.DMA,
    ],
)
def _gather_sc(table, idx, labs, out, labs_out, idx_v, rows_v, labs_v,
               gsem_a, gsem_b, ssem, lsem):
    wid = lax.axis_index("s") * NC + lax.axis_index("c")
    base = wid * BPW
    pltpu.sync_copy(idx.at[pl.ds(base, BPW)], idx_v)
    cp_a = pltpu.async_copy(table.at[idx_v.at[pl.ds(0, HALF)]],
                            rows_v.at[pl.ds(0, HALF)], gsem_a)
    cp_b = pltpu.async_copy(table.at[idx_v.at[pl.ds(HALF, HALF)]],
                            rows_v.at[pl.ds(HALF, HALF)], gsem_b)
    labs_cp = pltpu.async_copy(labs.at[idx_v], labs_v, lsem)

    nscat = [0]

    def _emit_scatters(chunk_lo, chunk_hi):
        for k in range(chunk_lo, chunk_hi):
            off = base + k * CHUNK

            @pl.when(off < B)
            def _(k=k, off=off):
                pltpu.async_copy(rows_v.at[pl.ds(k * CHUNK, CHUNK)],
                                 out.at[pl.ds(off, CHUNK)], ssem)
            nscat[0] += 1

    cp_a.wait()
    _emit_scatters(0, HALF // CHUNK)
    cp_b.wait()
    _emit_scatters(HALF // CHUNK, BPW // CHUNK)
    labs_cp.wait()
    for k in range(BPW // CHUNK):
        off = base + k * CHUNK

        @pl.when(off < B)
        def _(k=k, off=off):
            pltpu.sync_copy(labs_v.at[pl.ds(k * CHUNK, CHUNK)],
                            labs_out.at[pl.ds(off, CHUNK)])
            pltpu.make_async_copy(rows_v.at[pl.ds(k * CHUNK, CHUNK)],
                                  out.at[pl.ds(off, CHUNK)], ssem).wait()


def kernel(placeholder, table, labs):
    out, labs_out = _gather_sc(table, jnp.asarray(_IDX_PAD), labs)
    imgs = out.reshape(B, CHANNEL, H, W)
    indices = jnp.asarray(_INDICES)
    return (imgs, labs_out, indices)


# speedup vs baseline: 1.3920x; 1.0292x over previous
import functools

import jax
import jax.numpy as jnp
import numpy as np
from jax import lax
from jax.experimental import pallas as pl
from jax.experimental.pallas import tpu as pltpu
from jax.experimental.pallas import tpu_sc as plsc

IPC = 200
NUM_CLASSES = 100
CHANNEL, H, W = 3, 32, 32
N_PER_C = 10
DIM = CHANNEL * H * W
B = NUM_CLASSES * N_PER_C
ROWS = IPC * NUM_CLASSES

NC, NS = 2, 16
NW = NC * NS
B_PAD = 1024
BPW = B_PAD // NW
CHUNK = 8
NCHUNK = BPW // CHUNK


def _static_indices() -> np.ndarray:
    key = jax.random.key(42)
    parts = []
    for i in range(NUM_CLASSES):
        perm = jax.random.permutation(jax.random.fold_in(key, i), IPC)[:N_PER_C]
        parts.append(np.sort(np.asarray(perm)) + IPC * i)
    return np.concatenate(parts).astype(np.int32)


_INDICES = _static_indices()
_IDX_PAD = np.concatenate([_INDICES, np.zeros(B_PAD - B, np.int32)])

_mesh = plsc.VectorSubcoreMesh(core_axis_name="c", subcore_axis_name="s")


@functools.partial(
    pl.kernel,
    mesh=_mesh,
    out_type=(
        jax.ShapeDtypeStruct((B, DIM), jnp.float32),
        jax.ShapeDtypeStruct((B,), jnp.int32),
        jax.ShapeDtypeStruct((B,), jnp.int32),
    ),
    scratch_types=[
        pltpu.VMEM((BPW,), jnp.int32),
        pltpu.VMEM((BPW, DIM), jnp.float32),
        pltpu.VMEM((BPW,), jnp.int32),
        pltpu.SemaphoreType.DMA,
        pltpu.SemaphoreType.DMA,
    ],
)
def _gather_sc(table, idx, labs, out, labs_out, idx_out, idx_v, rows_v,
               labs_v, gsem, lsem):
    wid = lax.axis_index("s") * NC + lax.axis_index("c")
    base = wid * BPW
    pltpu.sync_copy(idx.at[pl.ds(base, BPW)], idx_v)
    rows_cp = pltpu.async_copy(table.at[idx_v], rows_v, gsem)
    labs_cp = pltpu.async_copy(labs.at[idx_v], labs_v, lsem)
    rows_cp.wait()
    labs_cp.wait()
    for k in range(NCHUNK):
        off = base + k * CHUNK

        @pl.when(off < B)
        def _(k=k, off=off):
            pltpu.sync_copy(rows_v.at[pl.ds(k * CHUNK, CHUNK)],
                            out.at[pl.ds(off, CHUNK)])
            pltpu.sync_copy(labs_v.at[pl.ds(k * CHUNK, CHUNK)],
                            labs_out.at[pl.ds(off, CHUNK)])
            pltpu.sync_copy(idx_v.at[pl.ds(k * CHUNK, CHUNK)],
                            idx_out.at[pl.ds(off, CHUNK)])


def kernel(placeholder, table, labs):
    out, labs_out, indices = _gather_sc(table, jnp.asarray(_IDX_PAD), labs)
    imgs = out.reshape(B, CHANNEL, H, W)
    return (imgs, labs_out, indices)
